# baseline (speedup 1.0000x reference)
import functools

import jax
import jax.numpy as jnp
from jax import lax
from jax.experimental import pallas as pl
from jax.experimental.pallas import tpu as pltpu
from jax.experimental.pallas import tpu_sc as plsc

R = 64
C = 32768
L = 16
NC = 2
NS = 16
WPC = 8
SC_ROWS = NC * WPC
TC_ROWS = R - SC_ROWS
UNROLL = 8
BLK = UNROLL * L
NCHUNK = 4
CHUNK = C // NCHUNK
BPC = CHUNK // BLK

_mesh = plsc.VectorSubcoreMesh(core_axis_name="c", subcore_axis_name="s")


def _row_body(buf):
    def body(b, carry):
        mvs = list(carry[0])
        mbs = list(carry[1])
        bb = lax.broadcast(b, (L,))
        base = b * BLK
        for u in range(UNROLL):
            v = buf[pl.ds(base + u * L, L)]
            gt = v > mvs[u]
            mvs[u] = jnp.maximum(v, mvs[u])
            mbs[u] = jnp.where(gt, bb, mbs[u])
        return tuple(mvs), tuple(mbs)
    return body


def _finalize(carry, lane):
    mvs, mbs = carry
    mv = mvs[0]
    mi = mbs[0] * BLK + lane
    for u in range(1, UNROLL):
        idx_u = mbs[u] * BLK + (lane + u * L)
        better = (mvs[u] > mv) | ((mvs[u] == mv) & (idx_u < mi))
        mv = jnp.where(better, mvs[u], mv)
        mi = jnp.where(better, idx_u, mi)
    for s in (8, 4, 2, 1):
        perm = lane ^ s
        mvp = mv.at[perm].get(mode="promise_in_bounds")
        mip = mi.at[perm].get(mode="promise_in_bounds")
        take = (mvp > mv) | ((mvp == mv) & (mip < mi))
        mv = jnp.where(take, mvp, mv)
        mi = jnp.where(take, mip, mi)
    return mv, mi


@functools.partial(
    pl.kernel,
    mesh=_mesh,
    out_type=[
        jax.ShapeDtypeStruct((SC_ROWS,), jnp.float32),
        jax.ShapeDtypeStruct((SC_ROWS,), jnp.int32),
    ],
    scratch_types=[
        pltpu.VMEM((C,), jnp.float32),
        pltpu.VMEM((L,), jnp.float32),
        pltpu.VMEM((L,), jnp.int32),
        pltpu.VMEM_SHARED((WPC * L,), jnp.float32),
        pltpu.VMEM_SHARED((WPC * L,), jnp.int32),
        pltpu.VMEM((WPC * L,), jnp.float32),
        pltpu.VMEM((WPC * L,), jnp.int32),
        pltpu.VMEM((L,), jnp.float32),
        pltpu.VMEM((L,), jnp.int32),
    ] + [pltpu.SemaphoreType.DMA] * NCHUNK,
)
def _topk1_sc(x_hbm, vals_hbm, idxs_hbm, buf, vstage, istage,
              shv, shi, gv, gi, ov, oi, *sems):
    cid = lax.axis_index("c")
    sid = lax.axis_index("s")
    active = sid < WPC
    row = TC_ROWS + cid * WPC + jnp.where(active, sid, 0)
    lane = lax.iota(jnp.int32, L)

    @pl.when(active)
    def _work():
        copies = [
            pltpu.async_copy(
                x_hbm.at[row, pl.ds(c * CHUNK, CHUNK)],
                buf.at[pl.ds(c * CHUNK, CHUNK)],
                sems[c])
            for c in range(NCHUNK)
        ]
        neg_inf = jnp.full((L,), -jnp.inf, dtype=jnp.float32)
        zero = jnp.zeros((L,), jnp.int32)
        carry = ((neg_inf,) * UNROLL, (zero,) * UNROLL)
        for c in range(NCHUNK):
            copies[c].wait()
            carry = lax.fori_loop(c * BPC, (c + 1) * BPC, _row_body(buf),
                                  carry)
        m, i = _finalize(carry, lane)

        own = lane == sid
        vstage[...] = jnp.where(own, m, jnp.zeros((L,), jnp.float32))
        istage[...] = jnp.where(own, i, zero)
        pltpu.sync_copy(vstage, shv.at[pl.ds(sid * L, L)])
        pltpu.sync_copy(istage, shi.at[pl.ds(sid * L, L)])

    plsc.subcore_barrier()

    @pl.when(sid == 0)
    def _aggregate():
        pltpu.sync_copy(shv, gv)
        pltpu.sync_copy(shi, gi)
        av = jnp.zeros((L,), jnp.float32)
        ai = jnp.zeros((L,), jnp.int32)
        for w in range(WPC):
            av = av + gv[pl.ds(w * L, L)]
            ai = ai + gi[pl.ds(w * L, L)]
        ov[...] = av
        oi[...] = ai
        cpv = pltpu.async_copy(ov.at[pl.ds(0, WPC)],
                               vals_hbm.at[pl.ds(cid * WPC, WPC)], sems[0])
        cpi = pltpu.async_copy(oi.at[pl.ds(0, WPC)],
                               idxs_hbm.at[pl.ds(cid * WPC, WPC)], sems[1])
        cpv.wait()
        cpi.wait()


def _tc_body(x_ref, v_ref, i_ref):
    xb = x_ref[...]
    m = jnp.max(xb, axis=1, keepdims=True)
    iota = lax.broadcasted_iota(jnp.int32, (TC_ROWS, C), 1)
    big = jnp.full((TC_ROWS, C), jnp.iinfo(jnp.int32).max, dtype=jnp.int32)
    idx = jnp.min(jnp.where(xb == m, iota, big), axis=1, keepdims=True)
    v_ref[...] = m
    i_ref[...] = idx


_topk1_tc = pl.pallas_call(
    _tc_body,
    grid=(1,),
    in_specs=[pl.BlockSpec((TC_ROWS, C), lambda i: (0, 0))],
    out_specs=[pl.BlockSpec((TC_ROWS, 1), lambda i: (0, 0)),
               pl.BlockSpec((TC_ROWS, 1), lambda i: (0, 0))],
    out_shape=[jax.ShapeDtypeStruct((TC_ROWS, 1), jnp.float32),
               jax.ShapeDtypeStruct((TC_ROWS, 1), jnp.int32)],
)


def _stitch_body(tv_ref, ti_ref, sv_ref, si_ref, v_ref, i_ref):
    v_ref[pl.ds(0, TC_ROWS), :] = tv_ref[...]
    i_ref[pl.ds(0, TC_ROWS), :] = ti_ref[...]
    v_ref[pl.ds(TC_ROWS, SC_ROWS), :] = sv_ref[...]
    i_ref[pl.ds(TC_ROWS, SC_ROWS), :] = si_ref[...]


_stitch = pl.pallas_call(
    _stitch_body,
    grid=(1,),
    in_specs=[pl.BlockSpec((TC_ROWS, 1), lambda i: (0, 0)),
              pl.BlockSpec((TC_ROWS, 1), lambda i: (0, 0)),
              pl.BlockSpec((SC_ROWS, 1), lambda i: (0, 0)),
              pl.BlockSpec((SC_ROWS, 1), lambda i: (0, 0))],
    out_specs=[pl.BlockSpec((R, 1), lambda i: (0, 0)),
               pl.BlockSpec((R, 1), lambda i: (0, 0))],
    out_shape=[jax.ShapeDtypeStruct((R, 1), jnp.float32),
               jax.ShapeDtypeStruct((R, 1), jnp.int32)],
)


def kernel(x):
    sc_v, sc_i = _topk1_sc(x)
    tc_v, tc_i = _topk1_tc(x)
    return _stitch(tc_v, tc_i, sc_v.reshape(SC_ROWS, 1),
                   sc_i.reshape(SC_ROWS, 1))


# speedup vs baseline: 1.2117x; 1.2117x over previous
import functools

import jax
import jax.numpy as jnp
from jax import lax
from jax.experimental import pallas as pl
from jax.experimental.pallas import tpu as pltpu
from jax.experimental.pallas import tpu_sc as plsc

R = 64
C = 32768
L = 16
NC = 2
NS = 16
NW = NC * NS
ROWS_PER_W = R // NW
UNROLL = 8
BLK = UNROLL * L
NCHUNK = 4
CHUNK = C // NCHUNK
BPC = CHUNK // BLK

_mesh = plsc.VectorSubcoreMesh(core_axis_name="c", subcore_axis_name="s")


def _row_body(buf):
    def body(b, carry):
        mvs = list(carry[0])
        mbs = list(carry[1])
        bb = lax.broadcast(b, (L,))
        base = b * BLK
        for u in range(UNROLL):
            v = buf[pl.ds(base + u * L, L)]
            gt = v > mvs[u]
            mvs[u] = jnp.maximum(v, mvs[u])
            mbs[u] = jnp.where(gt, bb, mbs[u])
        return tuple(mvs), tuple(mbs)
    return body


def _finalize(carry, lane):
    mvs, mbs = carry
    mv = mvs[0]
    mi = mbs[0] * BLK + lane
    for u in range(1, UNROLL):
        idx_u = mbs[u] * BLK + (lane + u * L)
        better = (mvs[u] > mv) | ((mvs[u] == mv) & (idx_u < mi))
        mv = jnp.where(better, mvs[u], mv)
        mi = jnp.where(better, idx_u, mi)
    for s in (8, 4, 2, 1):
        perm = lane ^ s
        mvp = mv.at[perm].get(mode="promise_in_bounds")
        mip = mi.at[perm].get(mode="promise_in_bounds")
        take = (mvp > mv) | ((mvp == mv) & (mip < mi))
        mv = jnp.where(take, mvp, mv)
        mi = jnp.where(take, mip, mi)
    return mv, mi


@functools.partial(
    pl.kernel,
    mesh=_mesh,
    out_type=[
        jax.ShapeDtypeStruct((R,), jnp.float32),
        jax.ShapeDtypeStruct((R,), jnp.int32),
    ],
    scratch_types=[
        pltpu.VMEM((C,), jnp.float32),
        pltpu.VMEM((C,), jnp.float32),
        pltpu.VMEM((L,), jnp.float32),
        pltpu.VMEM((L,), jnp.int32),
        pltpu.VMEM_SHARED((NS * L,), jnp.float32),
        pltpu.VMEM_SHARED((NS * L,), jnp.int32),
        pltpu.VMEM((NS * L,), jnp.float32),
        pltpu.VMEM((NS * L,), jnp.int32),
        pltpu.VMEM((2 * NS,), jnp.float32),
        pltpu.VMEM((2 * NS,), jnp.int32),
    ] + [pltpu.SemaphoreType.DMA] * (ROWS_PER_W * NCHUNK),
)
def _topk1_sc(x_hbm, vals_hbm, idxs_hbm, buf0, buf1, vstage, istage,
              shv, shi, gv, gi, ov, oi, *sems):
    cid = lax.axis_index("c")
    sid = lax.axis_index("s")
    wid = cid * NS + sid
    row0 = wid * ROWS_PER_W

    copies = []
    for r, buf in ((0, buf0), (1, buf1)):
        for c in range(NCHUNK):
            copies.append(pltpu.async_copy(
                x_hbm.at[row0 + r, pl.ds(c * CHUNK, CHUNK)],
                buf.at[pl.ds(c * CHUNK, CHUNK)],
                sems[r * NCHUNK + c]))

    lane = lax.iota(jnp.int32, L)
    neg_inf = jnp.full((L,), -jnp.inf, dtype=jnp.float32)
    zero = jnp.zeros((L,), jnp.int32)
    results = []
    for r, buf in ((0, buf0), (1, buf1)):
        carry = ((neg_inf,) * UNROLL, (zero,) * UNROLL)
        for c in range(NCHUNK):
            copies[r * NCHUNK + c].wait()
            carry = lax.fori_loop(c * BPC, (c + 1) * BPC, _row_body(buf),
                                  carry)
        results.append(_finalize(carry, lane))

    (m0, i0), (m1, i1) = results
    vstage[...] = jnp.where(lane == 0, m0, m1)
    istage[...] = jnp.where(lane == 0, i0, i1)
    pltpu.sync_copy(vstage, shv.at[pl.ds(sid * L, L)])
    pltpu.sync_copy(istage, shi.at[pl.ds(sid * L, L)])
    plsc.subcore_barrier()

    @pl.when(sid == 0)
    def _aggregate():
        pltpu.sync_copy(shv, gv)
        pltpu.sync_copy(shi, gi)
        half = lax.shift_right_logical(lane, 1)

        def compact(src, out_ref, init):
            for h in range(2):
                acc = init
                for j in range(NS // 2):
                    w = h * (NS // 2) + j
                    wv = src[pl.ds(w * L, L)]
                    perm = (lane - 2 * j) & (L - 1)
                    g = wv.at[perm].get(mode="promise_in_bounds")
                    acc = jnp.where(half == j, g, acc)
                out_ref[pl.ds(h * L, L)] = acc

        compact(gv, ov, jnp.zeros((L,), jnp.float32))
        compact(gi, oi, jnp.zeros((L,), jnp.int32))
        pltpu.sync_copy(ov, vals_hbm.at[pl.ds(cid * 2 * NS, 2 * NS)])
        pltpu.sync_copy(oi, idxs_hbm.at[pl.ds(cid * 2 * NS, 2 * NS)])


def kernel(x):
    values, indices = _topk1_sc(x)
    return values.reshape(R, 1), indices.reshape(R, 1)
